# Initial kernel scaffold; baseline (speedup 1.0000x reference)
import jax
import jax.numpy as jnp
from jax.experimental import pallas as pl


def kernel(edge_index, node_attr, edge_attr, batch, W_mpl, b_mpl, W1, b1, W2, b2, W3, b3, W4, b4):
    raise NotImplementedError("write your pallas kernel here")


# speedup vs baseline: 3.7304x; 3.7304x over previous
import functools

import jax
import jax.numpy as jnp
from jax import lax
from jax.experimental import pallas as pl
from jax.experimental.pallas import tpu as pltpu
from jax.experimental.pallas import tpu_sc as plsc

N = 10000
E = 320000
D = 128
DE = 16
H = 10
HP = 16
G = 64

CH = 128
NW = 32
KCW = 80
NCHUNK = KCW * NW
EP = NCHUNK * CH
NP = 10240
RPT = NP // 16

RB = 2048
NB = NP // RB

NBLK = 2000
EBLK = 5120


def _nproj_body(x_ref, w_ref, o_ref):
    o_ref[...] = jnp.dot(x_ref[...], w_ref[...],
                         preferred_element_type=jnp.float32)


def _eproj_body(x_ref, w_ref, b_ref, o_ref):
    o_ref[...] = jnp.dot(x_ref[...], w_ref[...],
                         preferred_element_type=jnp.float32) + b_ref[...]


_sc_mesh = plsc.VectorSubcoreMesh(core_axis_name="c", subcore_axis_name="s")


@functools.partial(
    pl.kernel,
    mesh=_sc_mesh,
    compiler_params=pltpu.CompilerParams(use_tc_tiling_on_sc=False),
    out_type=jax.ShapeDtypeStruct((2, NP, HP), jnp.float32),
    scratch_types=[
        pltpu.VMEM((KCW, CH), jnp.int32),
        pltpu.VMEM((KCW, CH), jnp.int32),
        pltpu.VMEM((CH, HP), jnp.float32),
        pltpu.VMEM((CH, HP), jnp.float32),
        pltpu.VMEM_SHARED((NP, HP), jnp.float32),
        pltpu.SemaphoreType.DMA,
        pltpu.SemaphoreType.DMA,
    ],
)
def _sc_msg(src_hbm, dst_hbm, eproj_hbm, nproj_hbm, zeros_hbm, out_hbm,
            src_v, dst_v, ep_v, nr_v, acc_sh, sem_e, sem_g):
    c = lax.axis_index("c")
    s = lax.axis_index("s")
    wid = s * 2 + c

    pltpu.sync_copy(zeros_hbm.at[pl.ds(s * RPT, RPT)],
                    acc_sh.at[pl.ds(s * RPT, RPT)])
    plsc.subcore_barrier()

    base = wid * KCW
    pltpu.sync_copy(src_hbm.at[pl.ds(base, KCW)], src_v)
    pltpu.sync_copy(dst_hbm.at[pl.ds(base, KCW)], dst_v)

    def chunk(j, carry):
        cp_e = pltpu.async_copy(eproj_hbm.at[pl.ds((base + j) * CH, CH)],
                                ep_v, sem_e)
        cp_g = pltpu.async_copy(nproj_hbm.at[src_v.at[j]], nr_v, sem_g)
        cp_e.wait()
        cp_g.wait()

        def row(i, c2):
            nr_v[i, :] = jnp.maximum(nr_v[i, :] + ep_v[i, :], 0.0)
            return c2

        lax.fori_loop(0, CH, row, 0)
        pltpu.sync_copy(nr_v, acc_sh.at[dst_v.at[j]], add=True)
        return carry

    lax.fori_loop(0, KCW, chunk, 0)
    plsc.subcore_barrier()
    pltpu.sync_copy(acc_sh.at[pl.ds(s * RPT, RPT)],
                    out_hbm.at[c].at[pl.ds(s * RPT, RPT)])


def _post_body(acc_ref, bat_ref, w1_ref, b1_ref, w2_ref, b2_ref,
               w3_ref, b3_ref, w4_ref, b4_ref, o_ref, seg_acc):
    i = pl.program_id(0)
    x = acc_ref[0] + acc_ref[1]
    bid = bat_ref[0, 0, :]
    x = jnp.tanh(jnp.dot(x, w1_ref[...],
                         preferred_element_type=jnp.float32) + b1_ref[...])
    x = jnp.tanh(jnp.dot(x, w2_ref[...],
                         preferred_element_type=jnp.float32) + b2_ref[...])
    onehot = (bid[None, :] == lax.broadcasted_iota(jnp.int32, (G, RB), 0)
              ).astype(jnp.float32)
    part = jnp.dot(onehot, x, preferred_element_type=jnp.float32)

    @pl.when(i == 0)
    def _():
        seg_acc[...] = jnp.zeros_like(seg_acc)

    seg_acc[...] += part

    @pl.when(i == NB - 1)
    def _():
        seg = seg_acc[...]
        y = jnp.tanh(jnp.dot(seg, w3_ref[...],
                             preferred_element_type=jnp.float32) + b3_ref[...])
        o_ref[...] = jnp.dot(y, w4_ref[...],
                             preferred_element_type=jnp.float32) + b4_ref[...]


def kernel(edge_index, node_attr, edge_attr, batch,
           W_mpl, b_mpl, W1, b1, W2, b2, W3, b3, W4, b4):
    f32 = jnp.float32

    wn = jnp.zeros((D, HP), f32).at[:, :H].set(W_mpl[:D])
    we = jnp.zeros((DE, HP), f32).at[:, :H].set(W_mpl[D:])
    bm = jnp.zeros((1, HP), f32).at[0, :H].set(b_mpl)
    w1p = jnp.zeros((HP, HP), f32).at[:H, :H].set(W1)
    b1p = jnp.zeros((1, HP), f32).at[0, :H].set(b1)
    w2p = jnp.zeros((HP, HP), f32).at[:H, :5].set(W2)
    b2p = jnp.zeros((1, HP), f32).at[0, :5].set(b2)
    w3p = jnp.zeros((HP, HP), f32).at[:5, :5].set(W3)
    b3p = jnp.zeros((1, HP), f32).at[0, :5].set(b3)
    w4p = jnp.zeros((HP, HP), f32).at[:5, :1].set(W4)
    b4p = jnp.zeros((1, HP), f32).at[0, :1].set(b4)

    src = jnp.concatenate(
        [edge_index[0], jnp.zeros((EP - E,), jnp.int32)]).reshape(NCHUNK, CH)
    dst = jnp.concatenate(
        [edge_index[1], jnp.full((EP - E,), N, jnp.int32)]).reshape(NCHUNK, CH)

    nproj = pl.pallas_call(
        _nproj_body,
        grid=(N // NBLK,),
        in_specs=[pl.BlockSpec((NBLK, D), lambda i: (i, 0)),
                  pl.BlockSpec((D, HP), lambda i: (0, 0))],
        out_specs=pl.BlockSpec((NBLK, HP), lambda i: (i, 0)),
        out_shape=jax.ShapeDtypeStruct((N, HP), f32),
    )(node_attr, wn)

    ea_pad = jnp.concatenate(
        [edge_attr, jnp.zeros((EP - E, DE), f32)])
    eproj = pl.pallas_call(
        _eproj_body,
        grid=(EP // EBLK,),
        in_specs=[pl.BlockSpec((EBLK, DE), lambda i: (i, 0)),
                  pl.BlockSpec((DE, HP), lambda i: (0, 0)),
                  pl.BlockSpec((1, HP), lambda i: (0, 0))],
        out_specs=pl.BlockSpec((EBLK, HP), lambda i: (i, 0)),
        out_shape=jax.ShapeDtypeStruct((EP, HP), f32),
    )(ea_pad, we, bm)

    zeros = jnp.zeros((NP, HP), f32)
    acc = _sc_msg(src, dst, eproj, nproj, zeros)

    batp = jnp.concatenate(
        [batch, jnp.full((NP - N,), G, jnp.int32)]).reshape(NB, 1, RB)

    out16 = pl.pallas_call(
        _post_body,
        grid=(NB,),
        in_specs=[pl.BlockSpec((2, RB, HP), lambda i: (0, i, 0)),
                  pl.BlockSpec((1, 1, RB), lambda i: (i, 0, 0)),
                  pl.BlockSpec((HP, HP), lambda i: (0, 0)),
                  pl.BlockSpec((1, HP), lambda i: (0, 0)),
                  pl.BlockSpec((HP, HP), lambda i: (0, 0)),
                  pl.BlockSpec((1, HP), lambda i: (0, 0)),
                  pl.BlockSpec((HP, HP), lambda i: (0, 0)),
                  pl.BlockSpec((1, HP), lambda i: (0, 0)),
                  pl.BlockSpec((HP, HP), lambda i: (0, 0)),
                  pl.BlockSpec((1, HP), lambda i: (0, 0))],
        out_specs=pl.BlockSpec((G, HP), lambda i: (0, 0)),
        out_shape=jax.ShapeDtypeStruct((G, HP), f32),
        scratch_shapes=[pltpu.VMEM((G, HP), f32)],
    )(acc, batp, w1p, b1p, w2p, b2p, w3p, b3p, w4p, b4p)

    return out16[:, :1]


# speedup vs baseline: 4.3921x; 1.1774x over previous
import functools

import jax
import jax.numpy as jnp
from jax import lax
from jax.experimental import pallas as pl
from jax.experimental.pallas import tpu as pltpu
from jax.experimental.pallas import tpu_sc as plsc

N = 10000
E = 320000
D = 128
DE = 16
H = 10
HP = 16
G = 64

CH = 128
NW = 32
KCW = 80
NCHUNK = KCW * NW
EP = NCHUNK * CH
NP = 10240
RPT = NP // 16

RB = 2048
NB = NP // RB

NBLK = 2000
EBLK = 5120


def _nproj_body(x_ref, w_ref, o_ref):
    o_ref[...] = jnp.dot(x_ref[...], w_ref[...],
                         preferred_element_type=jnp.float32)


def _eproj_body(x_ref, w_ref, b_ref, o_ref):
    o_ref[...] = jnp.dot(x_ref[...], w_ref[...],
                         preferred_element_type=jnp.float32) + b_ref[...]


_sc_mesh = plsc.VectorSubcoreMesh(core_axis_name="c", subcore_axis_name="s")


NBUF = 4
TOUT = KCW // NBUF


@functools.partial(
    pl.kernel,
    mesh=_sc_mesh,
    compiler_params=pltpu.CompilerParams(use_tc_tiling_on_sc=False),
    out_type=jax.ShapeDtypeStruct((2, NP, HP), jnp.float32),
    scratch_types=(
        [pltpu.VMEM((KCW, CH), jnp.int32)] * 2 +
        [pltpu.VMEM((CH, HP), jnp.float32)] * (3 * NBUF) +
        [pltpu.VMEM_SHARED((NP, HP), jnp.float32)] +
        [pltpu.SemaphoreType.DMA] * (3 * NBUF)
    ),
)
def _sc_msg(src_hbm, dst_hbm, eproj_hbm, nproj_hbm, zeros_hbm, out_hbm, *sc):
    src_v, dst_v = sc[0], sc[1]
    ep = sc[2:2 + NBUF]
    nr = sc[2 + NBUF:2 + 2 * NBUF]
    msg = sc[2 + 2 * NBUF:2 + 3 * NBUF]
    acc_sh = sc[2 + 3 * NBUF]
    esem = sc[3 + 3 * NBUF:3 + 4 * NBUF]
    gsem = sc[3 + 4 * NBUF:3 + 5 * NBUF]
    ssem = sc[3 + 5 * NBUF:3 + 6 * NBUF]

    c = lax.axis_index("c")
    s = lax.axis_index("s")
    wid = s * 2 + c

    pltpu.sync_copy(zeros_hbm.at[pl.ds(s * RPT, RPT)],
                    acc_sh.at[pl.ds(s * RPT, RPT)])
    plsc.subcore_barrier()

    base = wid * KCW
    pltpu.sync_copy(src_hbm.at[pl.ds(base, KCW)], src_v)
    pltpu.sync_copy(dst_hbm.at[pl.ds(base, KCW)], dst_v)

    def fetch(q, b):
        pltpu.async_copy(eproj_hbm.at[pl.ds((base + q) * CH, CH)],
                         ep[b], esem[b])
        pltpu.async_copy(nproj_hbm.at[src_v.at[q]], nr[b], gsem[b])

    def body(t, b, first, last):
        q = t * NBUF + b
        pltpu.make_async_copy(eproj_hbm.at[pl.ds(0, CH)], ep[b],
                              esem[b]).wait()
        pltpu.make_async_copy(nproj_hbm.at[src_v.at[0]], nr[b],
                              gsem[b]).wait()
        if not first:
            pltpu.make_async_copy(msg[b], acc_sh.at[dst_v.at[0]],
                                  ssem[b]).wait()

        def row(i, c2):
            msg[b][i, :] = jnp.maximum(nr[b][i, :] + ep[b][i, :], 0.0)
            return c2

        lax.fori_loop(0, CH, row, 0, unroll=8)
        pltpu.async_copy(msg[b], acc_sh.at[dst_v.at[q]], ssem[b], add=True)
        if not last:
            fetch(q + NBUF, b)

    for b in range(NBUF):
        fetch(b, b)
    for b in range(NBUF):
        body(0, b, first=True, last=False)

    def steady(t, carry):
        for b in range(NBUF):
            body(t, b, first=False, last=False)
        return carry

    lax.fori_loop(1, TOUT - 1, steady, 0)
    for b in range(NBUF):
        body(TOUT - 1, b, first=False, last=True)
    for b in range(NBUF):
        pltpu.make_async_copy(msg[b], acc_sh.at[dst_v.at[0]], ssem[b]).wait()

    plsc.subcore_barrier()
    pltpu.sync_copy(acc_sh.at[pl.ds(s * RPT, RPT)],
                    out_hbm.at[c].at[pl.ds(s * RPT, RPT)])


def _post_body(acc_ref, bat_ref, w1_ref, b1_ref, w2_ref, b2_ref,
               w3_ref, b3_ref, w4_ref, b4_ref, o_ref, seg_acc):
    i = pl.program_id(0)
    x = acc_ref[0] + acc_ref[1]
    bid = bat_ref[0, 0, :]
    x = jnp.tanh(jnp.dot(x, w1_ref[...],
                         preferred_element_type=jnp.float32) + b1_ref[...])
    x = jnp.tanh(jnp.dot(x, w2_ref[...],
                         preferred_element_type=jnp.float32) + b2_ref[...])
    onehot = (bid[None, :] == lax.broadcasted_iota(jnp.int32, (G, RB), 0)
              ).astype(jnp.float32)
    part = jnp.dot(onehot, x, preferred_element_type=jnp.float32)

    @pl.when(i == 0)
    def _():
        seg_acc[...] = jnp.zeros_like(seg_acc)

    seg_acc[...] += part

    @pl.when(i == NB - 1)
    def _():
        seg = seg_acc[...]
        y = jnp.tanh(jnp.dot(seg, w3_ref[...],
                             preferred_element_type=jnp.float32) + b3_ref[...])
        o_ref[...] = jnp.dot(y, w4_ref[...],
                             preferred_element_type=jnp.float32) + b4_ref[...]


def kernel(edge_index, node_attr, edge_attr, batch,
           W_mpl, b_mpl, W1, b1, W2, b2, W3, b3, W4, b4):
    f32 = jnp.float32

    wn = jnp.zeros((D, HP), f32).at[:, :H].set(W_mpl[:D])
    we = jnp.zeros((DE, HP), f32).at[:, :H].set(W_mpl[D:])
    bm = jnp.zeros((1, HP), f32).at[0, :H].set(b_mpl)
    w1p = jnp.zeros((HP, HP), f32).at[:H, :H].set(W1)
    b1p = jnp.zeros((1, HP), f32).at[0, :H].set(b1)
    w2p = jnp.zeros((HP, HP), f32).at[:H, :5].set(W2)
    b2p = jnp.zeros((1, HP), f32).at[0, :5].set(b2)
    w3p = jnp.zeros((HP, HP), f32).at[:5, :5].set(W3)
    b3p = jnp.zeros((1, HP), f32).at[0, :5].set(b3)
    w4p = jnp.zeros((HP, HP), f32).at[:5, :1].set(W4)
    b4p = jnp.zeros((1, HP), f32).at[0, :1].set(b4)

    src = jnp.concatenate(
        [edge_index[0], jnp.zeros((EP - E,), jnp.int32)]).reshape(NCHUNK, CH)
    dst = jnp.concatenate(
        [edge_index[1], jnp.full((EP - E,), N, jnp.int32)]).reshape(NCHUNK, CH)

    nproj = pl.pallas_call(
        _nproj_body,
        grid=(N // NBLK,),
        in_specs=[pl.BlockSpec((NBLK, D), lambda i: (i, 0)),
                  pl.BlockSpec((D, HP), lambda i: (0, 0))],
        out_specs=pl.BlockSpec((NBLK, HP), lambda i: (i, 0)),
        out_shape=jax.ShapeDtypeStruct((N, HP), f32),
    )(node_attr, wn)

    ea_pad = jnp.concatenate(
        [edge_attr, jnp.zeros((EP - E, DE), f32)])
    eproj = pl.pallas_call(
        _eproj_body,
        grid=(EP // EBLK,),
        in_specs=[pl.BlockSpec((EBLK, DE), lambda i: (i, 0)),
                  pl.BlockSpec((DE, HP), lambda i: (0, 0)),
                  pl.BlockSpec((1, HP), lambda i: (0, 0))],
        out_specs=pl.BlockSpec((EBLK, HP), lambda i: (i, 0)),
        out_shape=jax.ShapeDtypeStruct((EP, HP), f32),
    )(ea_pad, we, bm)

    zeros = jnp.zeros((NP, HP), f32)
    acc = _sc_msg(src, dst, eproj, nproj, zeros)

    batp = jnp.concatenate(
        [batch, jnp.full((NP - N,), G, jnp.int32)]).reshape(NB, 1, RB)

    out16 = pl.pallas_call(
        _post_body,
        grid=(NB,),
        in_specs=[pl.BlockSpec((2, RB, HP), lambda i: (0, i, 0)),
                  pl.BlockSpec((1, 1, RB), lambda i: (i, 0, 0)),
                  pl.BlockSpec((HP, HP), lambda i: (0, 0)),
                  pl.BlockSpec((1, HP), lambda i: (0, 0)),
                  pl.BlockSpec((HP, HP), lambda i: (0, 0)),
                  pl.BlockSpec((1, HP), lambda i: (0, 0)),
                  pl.BlockSpec((HP, HP), lambda i: (0, 0)),
                  pl.BlockSpec((1, HP), lambda i: (0, 0)),
                  pl.BlockSpec((HP, HP), lambda i: (0, 0)),
                  pl.BlockSpec((1, HP), lambda i: (0, 0))],
        out_specs=pl.BlockSpec((G, HP), lambda i: (0, 0)),
        out_shape=jax.ShapeDtypeStruct((G, HP), f32),
        scratch_shapes=[pltpu.VMEM((G, HP), f32)],
    )(acc, batp, w1p, b1p, w2p, b2p, w3p, b3p, w4p, b4p)

    return out16[:, :1]


# speedup vs baseline: 5.6618x; 1.2891x over previous
import functools

import jax
import jax.numpy as jnp
from jax import lax
from jax.experimental import pallas as pl
from jax.experimental.pallas import tpu as pltpu
from jax.experimental.pallas import tpu_sc as plsc

N = 10000
E = 320000
D = 128
DE = 16
H = 10
HP = 16
G = 64

CH = 128
NW = 32
KCW = 80
NCHUNK = KCW * NW
EP = NCHUNK * CH
NP = 10240
RPT = NP // 16

RB = 2048
NB = NP // RB

NBLK = 2000
EB8 = 320


def _nproj_body(x_ref, w_ref, o_ref):
    o_ref[...] = jnp.dot(x_ref[...], w_ref[...],
                         preferred_element_type=jnp.float32, precision=jax.lax.Precision.HIGHEST)


def _eproj_body(x_ref, w_ref, b_ref, o_ref):
    o_ref[...] = jnp.dot(x_ref[...], w_ref[...],
                         preferred_element_type=jnp.float32, precision=jax.lax.Precision.HIGHEST) + b_ref[...]


_sc_mesh = plsc.VectorSubcoreMesh(core_axis_name="c", subcore_axis_name="s")


NBUF = 4
TOUT = KCW // NBUF


@functools.partial(
    pl.kernel,
    mesh=_sc_mesh,
    compiler_params=pltpu.CompilerParams(use_tc_tiling_on_sc=False),
    out_type=jax.ShapeDtypeStruct((2, NP, HP), jnp.float32),
    scratch_types=(
        [pltpu.VMEM((KCW, CH), jnp.int32)] * 2 +
        [pltpu.VMEM((CH // 8, 128), jnp.float32)] * NBUF +
        [pltpu.VMEM((CH, HP), jnp.float32)] * (2 * NBUF) +
        [pltpu.VMEM_SHARED((NP, HP), jnp.float32)] +
        [pltpu.SemaphoreType.DMA] * (3 * NBUF)
    ),
)
def _sc_msg(src_hbm, dst_hbm, eproj_hbm, nproj_hbm, out_hbm, *sc):
    src_v, dst_v = sc[0], sc[1]
    ep = sc[2:2 + NBUF]
    nr = sc[2 + NBUF:2 + 2 * NBUF]
    msg = sc[2 + 2 * NBUF:2 + 3 * NBUF]
    acc_sh = sc[2 + 3 * NBUF]
    esem = sc[3 + 3 * NBUF:3 + 4 * NBUF]
    gsem = sc[3 + 4 * NBUF:3 + 5 * NBUF]
    ssem = sc[3 + 5 * NBUF:3 + 6 * NBUF]

    c = lax.axis_index("c")
    s = lax.axis_index("s")
    wid = s * 2 + c

    def zrow(i, c2):
        msg[0][i, :] = jnp.zeros((HP,), jnp.float32)
        return c2

    lax.fori_loop(0, CH, zrow, 0, unroll=8)

    def zcopy(k, c2):
        pltpu.sync_copy(msg[0], acc_sh.at[pl.ds(s * RPT + k * CH, CH)])
        return c2

    lax.fori_loop(0, RPT // CH, zcopy, 0)
    plsc.subcore_barrier()

    base = wid * KCW
    pltpu.sync_copy(src_hbm.at[pl.ds(base, KCW)], src_v)
    pltpu.sync_copy(dst_hbm.at[pl.ds(base, KCW)], dst_v)

    EPR = CH // 8

    def fetch(q, b):
        pltpu.async_copy(eproj_hbm.at[pl.ds((base + q) * EPR, EPR)],
                         ep[b], esem[b])
        pltpu.async_copy(nproj_hbm.at[src_v.at[q]], nr[b], gsem[b])

    def body(t, b, first, last):
        q = t * NBUF + b
        pltpu.make_async_copy(eproj_hbm.at[pl.ds(0, EPR)], ep[b],
                              esem[b]).wait()
        pltpu.make_async_copy(nproj_hbm.at[src_v.at[0]], nr[b],
                              gsem[b]).wait()
        if not first:
            pltpu.make_async_copy(msg[b], acc_sh.at[dst_v.at[0]],
                                  ssem[b]).wait()

        def row(r, c2):
            for u in range(8):
                i = 8 * r + u
                msg[b][i, :] = jnp.maximum(
                    nr[b][i, :] + ep[b][r, pl.ds(16 * u, 16)], 0.0)
            return c2

        lax.fori_loop(0, CH // 8, row, 0, unroll=2)
        pltpu.async_copy(msg[b], acc_sh.at[dst_v.at[q]], ssem[b], add=True)
        if not last:
            fetch(q + NBUF, b)

    for b in range(NBUF):
        fetch(b, b)
    for b in range(NBUF):
        body(0, b, first=True, last=False)

    def steady(t, carry):
        for b in range(NBUF):
            body(t, b, first=False, last=False)
        return carry

    lax.fori_loop(1, TOUT - 1, steady, 0)
    for b in range(NBUF):
        body(TOUT - 1, b, first=False, last=True)
    for b in range(NBUF):
        pltpu.make_async_copy(msg[b], acc_sh.at[dst_v.at[0]], ssem[b]).wait()

    plsc.subcore_barrier()
    pltpu.sync_copy(acc_sh.at[pl.ds(s * RPT, RPT)],
                    out_hbm.at[c].at[pl.ds(s * RPT, RPT)])


def _post_body(acc_ref, bat_ref, w1_ref, b1_ref, w2_ref, b2_ref,
               w3_ref, b3_ref, w4_ref, b4_ref, o_ref, seg_acc):
    i = pl.program_id(0)
    x = acc_ref[0] + acc_ref[1]
    bid = bat_ref[0, 0, :]
    x = jnp.tanh(jnp.dot(x, w1_ref[...],
                         preferred_element_type=jnp.float32, precision=jax.lax.Precision.HIGHEST) + b1_ref[...])
    x = jnp.tanh(jnp.dot(x, w2_ref[...],
                         preferred_element_type=jnp.float32, precision=jax.lax.Precision.HIGHEST) + b2_ref[...])
    onehot = (bid[None, :] == lax.broadcasted_iota(jnp.int32, (G, RB), 0)
              ).astype(jnp.float32)
    part = jnp.dot(onehot, x, preferred_element_type=jnp.float32, precision=jax.lax.Precision.HIGHEST)

    @pl.when(i == 0)
    def _():
        seg_acc[...] = jnp.zeros_like(seg_acc)

    seg_acc[...] += part

    @pl.when(i == NB - 1)
    def _():
        seg = seg_acc[...]
        y = jnp.tanh(jnp.dot(seg, w3_ref[...],
                             preferred_element_type=jnp.float32, precision=jax.lax.Precision.HIGHEST) + b3_ref[...])
        o_ref[...] = jnp.dot(y, w4_ref[...],
                             preferred_element_type=jnp.float32, precision=jax.lax.Precision.HIGHEST) + b4_ref[...]


def kernel(edge_index, node_attr, edge_attr, batch,
           W_mpl, b_mpl, W1, b1, W2, b2, W3, b3, W4, b4):
    f32 = jnp.float32

    wn = jnp.zeros((D, HP), f32).at[:, :H].set(W_mpl[:D])
    we = jnp.zeros((DE, HP), f32).at[:, :H].set(W_mpl[D:])
    bm = jnp.zeros((1, HP), f32).at[0, :H].set(b_mpl)
    w1p = jnp.zeros((HP, HP), f32).at[:H, :H].set(W1)
    b1p = jnp.zeros((1, HP), f32).at[0, :H].set(b1)
    w2p = jnp.zeros((HP, HP), f32).at[:H, :5].set(W2)
    b2p = jnp.zeros((1, HP), f32).at[0, :5].set(b2)
    w3p = jnp.zeros((HP, HP), f32).at[:5, :5].set(W3)
    b3p = jnp.zeros((1, HP), f32).at[0, :5].set(b3)
    w4p = jnp.zeros((HP, HP), f32).at[:5, :1].set(W4)
    b4p = jnp.zeros((1, HP), f32).at[0, :1].set(b4)

    src = jnp.concatenate(
        [edge_index[0], jnp.zeros((EP - E,), jnp.int32)]).reshape(NCHUNK, CH)
    dst = jnp.concatenate(
        [edge_index[1], jnp.full((EP - E,), N, jnp.int32)]).reshape(NCHUNK, CH)

    nproj = pl.pallas_call(
        _nproj_body,
        grid=(N // NBLK,),
        in_specs=[pl.BlockSpec((NBLK, D), lambda i: (i, 0)),
                  pl.BlockSpec((D, HP), lambda i: (0, 0))],
        out_specs=pl.BlockSpec((NBLK, HP), lambda i: (i, 0)),
        out_shape=jax.ShapeDtypeStruct((N, HP), f32),
    )(node_attr, wn)

    ea8 = edge_attr.reshape(E // 8, 8 * DE)
    we8 = jnp.kron(jnp.eye(8, dtype=f32), we)
    bm8 = jnp.tile(bm, (1, 8))
    last_blk = E // 8 // EB8 - 1
    eproj = pl.pallas_call(
        _eproj_body,
        grid=(EP // 8 // EB8,),
        in_specs=[pl.BlockSpec((EB8, 128),
                               lambda i: (jnp.minimum(i, last_blk), 0)),
                  pl.BlockSpec((128, 128), lambda i: (0, 0)),
                  pl.BlockSpec((1, 128), lambda i: (0, 0))],
        out_specs=pl.BlockSpec((EB8, 128), lambda i: (i, 0)),
        out_shape=jax.ShapeDtypeStruct((EP // 8, 128), f32),
    )(ea8, we8, bm8)

    acc = _sc_msg(src, dst, eproj, nproj)

    batp = jnp.concatenate(
        [batch, jnp.full((NP - N,), G, jnp.int32)]).reshape(NB, 1, RB)

    out16 = pl.pallas_call(
        _post_body,
        grid=(NB,),
        in_specs=[pl.BlockSpec((2, RB, HP), lambda i: (0, i, 0)),
                  pl.BlockSpec((1, 1, RB), lambda i: (i, 0, 0)),
                  pl.BlockSpec((HP, HP), lambda i: (0, 0)),
                  pl.BlockSpec((1, HP), lambda i: (0, 0)),
                  pl.BlockSpec((HP, HP), lambda i: (0, 0)),
                  pl.BlockSpec((1, HP), lambda i: (0, 0)),
                  pl.BlockSpec((HP, HP), lambda i: (0, 0)),
                  pl.BlockSpec((1, HP), lambda i: (0, 0)),
                  pl.BlockSpec((HP, HP), lambda i: (0, 0)),
                  pl.BlockSpec((1, HP), lambda i: (0, 0))],
        out_specs=pl.BlockSpec((G, HP), lambda i: (0, 0)),
        out_shape=jax.ShapeDtypeStruct((G, HP), f32),
        scratch_shapes=[pltpu.VMEM((G, HP), f32)],
    )(acc, batp, w1p, b1p, w2p, b2p, w3p, b3p, w4p, b4p)

    return out16[:, :1]


# speedup vs baseline: 6.2406x; 1.1022x over previous
import functools

import jax
import jax.numpy as jnp
from jax import lax
from jax.experimental import pallas as pl
from jax.experimental.pallas import tpu as pltpu
from jax.experimental.pallas import tpu_sc as plsc

N = 10000
E = 320000
D = 128
DE = 16
H = 10
HP = 16
G = 64

CH = 128
NW = 32
KCW = 80
NCHUNK = KCW * NW
EP = NCHUNK * CH
NP = 10240
RPT = NP // 16

RB = 2048
NB = NP // RB

NBLK = 2000
EBLK = 20480
DR = EBLK // 8
G_ROWS = 512
QPB = DR // 16


def _nproj_body(x_ref, w_ref, o_ref):
    o_ref[...] = jnp.dot(x_ref[...], w_ref[...],
                         preferred_element_type=jnp.float32, precision=jax.lax.Precision.HIGHEST)


def _eproj_body(x_ref, w_ref, b_ref, o_ref):
    i = pl.program_id(0)
    thr = jnp.clip(E // DR - 8 * i, 0, 8) * HP
    lmask = lax.broadcasted_iota(jnp.int32, (G_ROWS, 128), 1) < thr
    for g in range(DR // G_ROWS):
        parts = [x_ref[pl.ds(DR * j + G_ROWS * g, G_ROWS), :]
                 for j in range(8)]
        x128 = jnp.concatenate(parts, axis=1)
        y = jnp.dot(x128, w_ref[...], preferred_element_type=jnp.float32,
                    precision=jax.lax.Precision.HIGHEST) + b_ref[...]
        o_ref[pl.ds(G_ROWS * g, G_ROWS), :] = jnp.where(lmask, y, 0.0)


_sc_mesh = plsc.VectorSubcoreMesh(core_axis_name="c", subcore_axis_name="s")


NBUF = 4
TOUT = KCW // NBUF


@functools.partial(
    pl.kernel,
    mesh=_sc_mesh,
    compiler_params=pltpu.CompilerParams(use_tc_tiling_on_sc=False),
    out_type=jax.ShapeDtypeStruct((2, NP, HP), jnp.float32),
    scratch_types=(
        [pltpu.VMEM((KCW, CH), jnp.int32)] * 2 +
        [pltpu.VMEM((CH // 8, 128), jnp.float32)] * NBUF +
        [pltpu.VMEM((CH, HP), jnp.float32)] * (2 * NBUF) +
        [pltpu.VMEM_SHARED((NP, HP), jnp.float32)] +
        [pltpu.SemaphoreType.DMA] * (3 * NBUF)
    ),
)
def _sc_msg(src_hbm, dst_hbm, eproj_hbm, nproj_hbm, out_hbm, *sc):
    src_v, dst_v = sc[0], sc[1]
    ep = sc[2:2 + NBUF]
    nr = sc[2 + NBUF:2 + 2 * NBUF]
    msg = sc[2 + 2 * NBUF:2 + 3 * NBUF]
    acc_sh = sc[2 + 3 * NBUF]
    esem = sc[3 + 3 * NBUF:3 + 4 * NBUF]
    gsem = sc[3 + 4 * NBUF:3 + 5 * NBUF]
    ssem = sc[3 + 5 * NBUF:3 + 6 * NBUF]

    c = lax.axis_index("c")
    s = lax.axis_index("s")
    wid = s * 2 + c

    def zrow(i, c2):
        msg[0][i, :] = jnp.zeros((HP,), jnp.float32)
        return c2

    lax.fori_loop(0, CH, zrow, 0, unroll=8)

    def zcopy(k, c2):
        pltpu.sync_copy(msg[0], acc_sh.at[pl.ds(s * RPT + k * CH, CH)])
        return c2

    lax.fori_loop(0, RPT // CH, zcopy, 0)
    plsc.subcore_barrier()

    base = wid * KCW
    pltpu.sync_copy(src_hbm.at[pl.ds(base, KCW)], src_v)
    pltpu.sync_copy(dst_hbm.at[pl.ds(base, KCW)], dst_v)

    EPR = CH // 8

    def fetch(q, b):
        pltpu.async_copy(eproj_hbm.at[pl.ds((base + q) * EPR, EPR)],
                         ep[b], esem[b])
        pltpu.async_copy(nproj_hbm.at[src_v.at[q]], nr[b], gsem[b])

    def body(t, b, first, last):
        q = t * NBUF + b
        pltpu.make_async_copy(eproj_hbm.at[pl.ds(0, EPR)], ep[b],
                              esem[b]).wait()
        pltpu.make_async_copy(nproj_hbm.at[src_v.at[0]], nr[b],
                              gsem[b]).wait()
        if not first:
            pltpu.make_async_copy(msg[b], acc_sh.at[dst_v.at[0]],
                                  ssem[b]).wait()

        def row(r, c2):
            for u in range(8):
                i = 8 * r + u
                msg[b][i, :] = jnp.maximum(
                    nr[b][i, :] + ep[b][r, pl.ds(16 * u, 16)], 0.0)
            return c2

        lax.fori_loop(0, CH // 8, row, 0, unroll=2)
        pltpu.async_copy(msg[b], acc_sh.at[dst_v.at[q]], ssem[b], add=True)
        if not last:
            fetch(q + NBUF, b)

    for b in range(NBUF):
        fetch(b, b)
    for b in range(NBUF):
        body(0, b, first=True, last=False)

    def steady(t, carry):
        for b in range(NBUF):
            body(t, b, first=False, last=False)
        return carry

    lax.fori_loop(1, TOUT - 1, steady, 0)
    for b in range(NBUF):
        body(TOUT - 1, b, first=False, last=True)
    for b in range(NBUF):
        pltpu.make_async_copy(msg[b], acc_sh.at[dst_v.at[0]], ssem[b]).wait()

    plsc.subcore_barrier()
    pltpu.sync_copy(acc_sh.at[pl.ds(s * RPT, RPT)],
                    out_hbm.at[c].at[pl.ds(s * RPT, RPT)])


def _post_body(acc_ref, bat_ref, w1_ref, b1_ref, w2_ref, b2_ref,
               w3_ref, b3_ref, w4_ref, b4_ref, o_ref, seg_acc):
    i = pl.program_id(0)
    x = acc_ref[0] + acc_ref[1]
    bid = bat_ref[0, 0, :]
    x = jnp.tanh(jnp.dot(x, w1_ref[...],
                         preferred_element_type=jnp.float32, precision=jax.lax.Precision.HIGHEST) + b1_ref[...])
    x = jnp.tanh(jnp.dot(x, w2_ref[...],
                         preferred_element_type=jnp.float32, precision=jax.lax.Precision.HIGHEST) + b2_ref[...])
    onehot = (bid[None, :] == lax.broadcasted_iota(jnp.int32, (G, RB), 0)
              ).astype(jnp.float32)
    part = jnp.dot(onehot, x, preferred_element_type=jnp.float32, precision=jax.lax.Precision.HIGHEST)

    @pl.when(i == 0)
    def _():
        seg_acc[...] = jnp.zeros_like(seg_acc)

    seg_acc[...] += part

    @pl.when(i == NB - 1)
    def _():
        seg = seg_acc[...]
        y = jnp.tanh(jnp.dot(seg, w3_ref[...],
                             preferred_element_type=jnp.float32, precision=jax.lax.Precision.HIGHEST) + b3_ref[...])
        o_ref[...] = jnp.dot(y, w4_ref[...],
                             preferred_element_type=jnp.float32, precision=jax.lax.Precision.HIGHEST) + b4_ref[...]


def kernel(edge_index, node_attr, edge_attr, batch,
           W_mpl, b_mpl, W1, b1, W2, b2, W3, b3, W4, b4):
    f32 = jnp.float32

    wn = jnp.zeros((D, HP), f32).at[:, :H].set(W_mpl[:D])
    we = jnp.zeros((DE, HP), f32).at[:, :H].set(W_mpl[D:])
    bm = jnp.zeros((1, HP), f32).at[0, :H].set(b_mpl)
    w1p = jnp.zeros((HP, HP), f32).at[:H, :H].set(W1)
    b1p = jnp.zeros((1, HP), f32).at[0, :H].set(b1)
    w2p = jnp.zeros((HP, HP), f32).at[:H, :5].set(W2)
    b2p = jnp.zeros((1, HP), f32).at[0, :5].set(b2)
    w3p = jnp.zeros((HP, HP), f32).at[:5, :5].set(W3)
    b3p = jnp.zeros((1, HP), f32).at[0, :5].set(b3)
    w4p = jnp.zeros((HP, HP), f32).at[:5, :1].set(W4)
    b4p = jnp.zeros((1, HP), f32).at[0, :1].set(b4)

    def _chunked(ids, fill):
        p = jnp.concatenate([ids, jnp.full((EP - E,), fill, jnp.int32)])
        return (p.reshape(EP // EBLK, 8, QPB, 16)
                .transpose(0, 2, 3, 1).reshape(NCHUNK, CH))

    src = _chunked(edge_index[0], 0)
    dst = _chunked(edge_index[1], N)

    nproj = pl.pallas_call(
        _nproj_body,
        grid=(N // NBLK,),
        in_specs=[pl.BlockSpec((NBLK, D), lambda i: (i, 0)),
                  pl.BlockSpec((D, HP), lambda i: (0, 0))],
        out_specs=pl.BlockSpec((NBLK, HP), lambda i: (i, 0)),
        out_shape=jax.ShapeDtypeStruct((N, HP), f32),
    )(node_attr, wn)

    we8 = jnp.kron(jnp.eye(8, dtype=f32), we)
    bm8 = jnp.tile(bm, (1, 8))
    eproj = pl.pallas_call(
        _eproj_body,
        grid=(EP // EBLK,),
        in_specs=[pl.BlockSpec((EBLK, DE), lambda i: (i, 0)),
                  pl.BlockSpec((128, 128), lambda i: (0, 0)),
                  pl.BlockSpec((1, 128), lambda i: (0, 0))],
        out_specs=pl.BlockSpec((DR, 128), lambda i: (i, 0)),
        out_shape=jax.ShapeDtypeStruct((EP // 8, 128), f32),
    )(edge_attr, we8, bm8)

    acc = _sc_msg(src, dst, eproj, nproj)

    batp = jnp.concatenate(
        [batch, jnp.full((NP - N,), G, jnp.int32)]).reshape(NB, 1, RB)

    out16 = pl.pallas_call(
        _post_body,
        grid=(NB,),
        in_specs=[pl.BlockSpec((2, RB, HP), lambda i: (0, i, 0)),
                  pl.BlockSpec((1, 1, RB), lambda i: (i, 0, 0)),
                  pl.BlockSpec((HP, HP), lambda i: (0, 0)),
                  pl.BlockSpec((1, HP), lambda i: (0, 0)),
                  pl.BlockSpec((HP, HP), lambda i: (0, 0)),
                  pl.BlockSpec((1, HP), lambda i: (0, 0)),
                  pl.BlockSpec((HP, HP), lambda i: (0, 0)),
                  pl.BlockSpec((1, HP), lambda i: (0, 0)),
                  pl.BlockSpec((HP, HP), lambda i: (0, 0)),
                  pl.BlockSpec((1, HP), lambda i: (0, 0))],
        out_specs=pl.BlockSpec((G, HP), lambda i: (0, 0)),
        out_shape=jax.ShapeDtypeStruct((G, HP), f32),
        scratch_shapes=[pltpu.VMEM((G, HP), f32)],
    )(acc, batp, w1p, b1p, w2p, b2p, w3p, b3p, w4p, b4p)

    return out16[:, :1]
